# baseline (speedup 1.0000x reference)
import functools

import jax
import jax.numpy as jnp
from jax import lax
from jax.experimental import pallas as pl
from jax.experimental.pallas import tpu as pltpu
from jax.experimental.pallas import tpu_sc as plsc

N = 10000
E = 160000
IN_C = 16
OUT_C = 16
D_EDGE = 8
H = 100
HP = 128
KC = IN_C * OUT_C
LW = 128
AW = 32

NC = 2
NS = 16
NW = NC * NS
E_PAD = 163840
EPW = E_PAD // NW
CH = 128
NCHUNK = EPW // CH
GRP = 4
GRP2 = 2
N_PAD = 10240
RPW = N_PAD // NS

_sc_mesh = plsc.VectorSubcoreMesh(core_axis_name="c", subcore_axis_name="s")


@functools.partial(
    pl.kernel,
    mesh=_sc_mesh,
    out_type=jax.ShapeDtypeStruct((E_PAD, LW), jnp.float32),
    scratch_types=[
        pltpu.VMEM((NCHUNK, CH), jnp.int32),
        pltpu.VMEM((GRP, CH, LW), jnp.float32),
        pltpu.SemaphoreType.DMA,
    ],
)
def _sc_gather(x_hbm, src_hbm, xj_hbm, idx_v, bufs_v, sem):
    c = lax.axis_index("c")
    s = lax.axis_index("s")
    wid = c * NS + s
    base = wid * EPW
    pltpu.sync_copy(src_hbm.at[pl.ds(wid * NCHUNK, NCHUNK)], idx_v)
    for g in range(NCHUNK // GRP):
        for k in range(GRP):
            j = g * GRP + k
            pltpu.async_copy(x_hbm.at[idx_v.at[j]], bufs_v.at[k], sem)
        for k in range(GRP):
            j = g * GRP + k
            pltpu.make_async_copy(x_hbm.at[idx_v.at[j]], bufs_v.at[k], sem).wait()
        for k in range(GRP):
            j = g * GRP + k
            pltpu.sync_copy(bufs_v.at[k], xj_hbm.at[pl.ds(base + j * CH, CH)])


@functools.partial(
    pl.kernel,
    mesh=_sc_mesh,
    out_type=jax.ShapeDtypeStruct((NC * N_PAD, LW), jnp.float32),
    scratch_types=[
        pltpu.VMEM((NCHUNK, CH), jnp.int32),
        pltpu.VMEM((GRP2, CH, LW), jnp.float32),
        pltpu.VMEM_SHARED((N_PAD, LW), jnp.float32),
        pltpu.SemaphoreType.DMA,
    ],
)
def _sc_scatter(msg_hbm, dst_hbm, acc_out, idx_v, bufs_v, acc_sh, sem):
    c = lax.axis_index("c")
    s = lax.axis_index("s")
    wid = c * NS + s
    base = wid * EPW
    pltpu.sync_copy(dst_hbm.at[pl.ds(wid * NCHUNK, NCHUNK)], idx_v)

    def zb(i, carry):
        for q in range(LW // 16):
            bufs_v[0, i, pl.ds(q * 16, 16)] = jnp.zeros((16,), jnp.float32)
        return carry

    lax.fori_loop(0, CH, zb, 0)
    for t in range(RPW // CH):
        pltpu.sync_copy(bufs_v.at[0], acc_sh.at[pl.ds(s * RPW + t * CH, CH)])
    plsc.subcore_barrier()

    for g in range(NCHUNK // GRP2):
        for k in range(GRP2):
            j = g * GRP2 + k
            pltpu.async_copy(msg_hbm.at[pl.ds(base + j * CH, CH)],
                             bufs_v.at[k], sem)
        for k in range(GRP2):
            j = g * GRP2 + k
            pltpu.make_async_copy(msg_hbm.at[pl.ds(base + j * CH, CH)],
                                  bufs_v.at[k], sem).wait()
        for k in range(GRP2):
            j = g * GRP2 + k
            pltpu.sync_copy(bufs_v.at[k], acc_sh.at[idx_v.at[j]], add=True)

    plsc.subcore_barrier()
    pltpu.sync_copy(acc_sh.at[pl.ds(s * RPW, RPW)],
                    acc_out.at[pl.ds(c * N_PAD + s * RPW, RPW)])


BE = 2048
GRID = E_PAD // BE


def _mlp_body(ea_ref, xj_ref, w1, b1, w2, b2, w3, b3, w4, b4, r_ref, s_ref,
              msg_ref):
    f32 = jnp.float32
    bf16 = jnp.bfloat16

    def mm(a, b):
        return jnp.dot(a.astype(bf16), b.astype(bf16), preferred_element_type=f32)

    h = jnp.maximum(mm(ea_ref[...], w1[...]) + b1[...], 0.0)
    h = jnp.maximum(mm(h, w2[...]) + b2[...], 0.0)
    h = jnp.maximum(mm(h, w3[...]) + b3[...], 0.0)
    z = mm(h, w4[...]) + b4[...]
    xj = xj_ref[:, pl.ds(0, IN_C)]
    xe = jnp.dot(xj, r_ref[...], preferred_element_type=f32)
    msg = mm(z * xe, s_ref[...])
    lane = lax.broadcasted_iota(jnp.int32, (BE, LW), 1)
    msg_ref[...] = jnp.where(lane == IN_C, 1.0,
                             jnp.pad(msg, ((0, 0), (0, LW - OUT_C))))


_mlp_call = pl.pallas_call(
    _mlp_body,
    grid=(GRID,),
    in_specs=[
        pl.BlockSpec((BE, D_EDGE), lambda i: (i, 0)),
        pl.BlockSpec((BE, LW), lambda i: (i, 0)),
        pl.BlockSpec((D_EDGE, HP), lambda i: (0, 0)),
        pl.BlockSpec((1, HP), lambda i: (0, 0)),
        pl.BlockSpec((HP, HP), lambda i: (0, 0)),
        pl.BlockSpec((1, HP), lambda i: (0, 0)),
        pl.BlockSpec((HP, HP), lambda i: (0, 0)),
        pl.BlockSpec((1, HP), lambda i: (0, 0)),
        pl.BlockSpec((HP, KC), lambda i: (0, 0)),
        pl.BlockSpec((1, KC), lambda i: (0, 0)),
        pl.BlockSpec((IN_C, KC), lambda i: (0, 0)),
        pl.BlockSpec((KC, OUT_C), lambda i: (0, 0)),
    ],
    out_specs=pl.BlockSpec((BE, LW), lambda i: (i, 0)),
    out_shape=jax.ShapeDtypeStruct((E_PAD, LW), jnp.float32),
)


def _final_body(acc_ref, x_ref, rw, bias_ref, out_ref):
    a0 = acc_ref[pl.ds(0, N), pl.ds(0, OUT_C)]
    a1 = acc_ref[pl.ds(N_PAD, N), pl.ds(0, OUT_C)]
    c0 = acc_ref[pl.ds(0, N), pl.ds(IN_C, 1)]
    c1 = acc_ref[pl.ds(N_PAD, N), pl.ds(IN_C, 1)]
    cnt = jnp.maximum(c0 + c1, 1.0)
    aggr = (a0 + a1) / cnt
    out_ref[...] = aggr + jnp.dot(x_ref[...], rw[...],
                                  preferred_element_type=jnp.float32) + bias_ref[...]


_final_call = pl.pallas_call(
    _final_body,
    out_shape=jax.ShapeDtypeStruct((N, OUT_C), jnp.float32),
)


def kernel(x, edge_index, edge_attr, W1, b1, W2, b2, W3, b3, W4, b4, root_w, bias):
    f32 = jnp.float32
    pad_e = E_PAD - E
    src = edge_index[0].astype(jnp.int32)
    dst = edge_index[1].astype(jnp.int32)
    src_p = jnp.concatenate([src, jnp.zeros((pad_e,), jnp.int32)]
                            ).reshape(E_PAD // CH, CH)
    dst_p = jnp.concatenate([dst, jnp.full((pad_e,), N, jnp.int32)]
                            ).reshape(E_PAD // CH, CH)
    ea_p = jnp.pad(edge_attr, ((0, pad_e), (0, 0)))
    x128 = jnp.pad(x, ((0, 0), (0, LW - IN_C)))

    W1p = jnp.pad(W1, ((0, 0), (0, HP - H)))
    b1p = jnp.pad(b1, (0, HP - H)).reshape(1, HP)
    W2p = jnp.pad(W2, ((0, HP - H), (0, HP - H)))
    b2p = jnp.pad(b2, (0, HP - H)).reshape(1, HP)
    W3p = jnp.pad(W3, ((0, HP - H), (0, HP - H)))
    b3p = jnp.pad(b3, (0, HP - H)).reshape(1, HP)
    W4p = jnp.pad(W4, ((0, HP - H), (0, 0)))
    b4p = b4.reshape(1, KC)

    R = jnp.repeat(jnp.eye(IN_C, dtype=f32), OUT_C, axis=1)
    S = jnp.tile(jnp.eye(OUT_C, dtype=f32), (IN_C, 1))

    x_j = _sc_gather(x128, src_p)
    msg = _mlp_call(ea_p, x_j, W1p, b1p, W2p, b2p, W3p, b3p, W4p, b4p, R, S)
    acc = _sc_scatter(msg, dst_p)
    return _final_call(acc, x, root_w, bias)


# speedup vs baseline: 1.5584x; 1.5584x over previous
import functools

import jax
import jax.numpy as jnp
from jax import lax
from jax.experimental import pallas as pl
from jax.experimental.pallas import tpu as pltpu
from jax.experimental.pallas import tpu_sc as plsc

N = 10000
E = 160000
IN_C = 16
OUT_C = 16
D_EDGE = 8
H = 100
HP = 128
KC = IN_C * OUT_C
LW = 128

NC = 2
NS = 16
NW = NC * NS
E_PAD = 163840
EPW = E_PAD // NW
CH = 128
NCHUNK = EPW // CH
N_PAD = 10240
RPW = N_PAD // NS

_sc_mesh = plsc.VectorSubcoreMesh(core_axis_name="c", subcore_axis_name="s")


@functools.partial(
    pl.kernel,
    mesh=_sc_mesh,
    out_type=jax.ShapeDtypeStruct((E_PAD, IN_C), jnp.bfloat16),
    scratch_types=[
        pltpu.VMEM((NCHUNK, CH), jnp.int32),
        pltpu.VMEM((EPW, IN_C), jnp.bfloat16),
        pltpu.SemaphoreType.DMA,
    ],
    compiler_params=pltpu.CompilerParams(use_tc_tiling_on_sc=False),
)
def _sc_gather(x_hbm, src_hbm, xj_hbm, idx_v, rows_v, sem):
    c = lax.axis_index("c")
    s = lax.axis_index("s")
    wid = c * NS + s
    pltpu.sync_copy(src_hbm.at[pl.ds(wid * NCHUNK, NCHUNK)], idx_v)

    def issue(j, carry):
        pltpu.async_copy(x_hbm.at[idx_v.at[j]], rows_v.at[pl.ds(j * CH, CH)], sem)
        return carry

    lax.fori_loop(0, NCHUNK, issue, 0)

    def drain(j, carry):
        pltpu.make_async_copy(
            x_hbm.at[idx_v.at[j]], rows_v.at[pl.ds(j * CH, CH)], sem
        ).wait()
        return carry

    lax.fori_loop(0, NCHUNK, drain, 0)
    pltpu.sync_copy(rows_v, xj_hbm.at[pl.ds(wid * EPW, EPW)])


@functools.partial(
    pl.kernel,
    mesh=_sc_mesh,
    out_type=(
        jax.ShapeDtypeStruct((NC * N_PAD, OUT_C), jnp.float32),
        jax.ShapeDtypeStruct((NC * N_PAD, OUT_C), jnp.float32),
    ),
    scratch_types=[
        pltpu.VMEM((NCHUNK, CH), jnp.int32),
        pltpu.VMEM((EPW, OUT_C), jnp.float32),
        pltpu.VMEM((CH, OUT_C), jnp.float32),
        pltpu.VMEM_SHARED((N_PAD, OUT_C), jnp.float32),
        pltpu.VMEM_SHARED((N_PAD, OUT_C), jnp.float32),
    ],
    compiler_params=pltpu.CompilerParams(use_tc_tiling_on_sc=False),
)
def _sc_scatter(msg_hbm, dst_hbm, zeros_hbm, ones_hbm,
                acc_out, cnt_out, idx_v, msg_v, ones_v, acc_sh, cnt_sh):
    c = lax.axis_index("c")
    s = lax.axis_index("s")
    wid = c * NS + s
    pltpu.sync_copy(dst_hbm.at[pl.ds(wid * NCHUNK, NCHUNK)], idx_v)
    pltpu.sync_copy(msg_hbm.at[pl.ds(wid * EPW, EPW)], msg_v)
    pltpu.sync_copy(ones_hbm, ones_v)

    @pl.when(s == 0)
    def _():
        pltpu.sync_copy(zeros_hbm, acc_sh)
        pltpu.sync_copy(zeros_hbm, cnt_sh)

    plsc.subcore_barrier()

    def body(j, carry):
        row = idx_v.at[j]
        pltpu.sync_copy(msg_v.at[pl.ds(j * CH, CH)], acc_sh.at[row], add=True)
        pltpu.sync_copy(ones_v, cnt_sh.at[row], add=True)
        return carry

    lax.fori_loop(0, NCHUNK, body, 0)
    plsc.subcore_barrier()

    out_off = c * N_PAD + s * RPW
    pltpu.sync_copy(acc_sh.at[pl.ds(s * RPW, RPW)], acc_out.at[pl.ds(out_off, RPW)])
    pltpu.sync_copy(cnt_sh.at[pl.ds(s * RPW, RPW)], cnt_out.at[pl.ds(out_off, RPW)])


BE = 2048
GRID = E_PAD // BE


def _mlp_body(ea_ref, xj_ref, w1, b1, w2, b2, w3, b3, w4, b4, r_ref, s_ref,
              msg_ref):
    f32 = jnp.float32
    bf16 = jnp.bfloat16

    def mm(a, b):
        return jnp.dot(a.astype(bf16), b.astype(bf16), preferred_element_type=f32)

    h1 = lax.dot_general(ea_ref[...].astype(bf16), w1[...].astype(bf16),
                         (((0,), (0,)), ((), ())),
                         preferred_element_type=f32)
    h = jnp.maximum(h1 + b1[...], 0.0)
    h = jnp.maximum(mm(h, w2[...]) + b2[...], 0.0)
    h = jnp.maximum(mm(h, w3[...]) + b3[...], 0.0)
    z = mm(h, w4[...]) + b4[...]
    xe = jnp.dot(xj_ref[...], r_ref[...].astype(bf16),
                 preferred_element_type=f32)
    msg_ref[...] = mm(z * xe, s_ref[...])


_mlp_call = pl.pallas_call(
    _mlp_body,
    grid=(GRID,),
    in_specs=[
        pl.BlockSpec((D_EDGE, BE), lambda i: (0, i)),
        pl.BlockSpec((BE, IN_C), lambda i: (i, 0)),
        pl.BlockSpec((D_EDGE, HP), lambda i: (0, 0)),
        pl.BlockSpec((1, HP), lambda i: (0, 0)),
        pl.BlockSpec((HP, HP), lambda i: (0, 0)),
        pl.BlockSpec((1, HP), lambda i: (0, 0)),
        pl.BlockSpec((HP, HP), lambda i: (0, 0)),
        pl.BlockSpec((1, HP), lambda i: (0, 0)),
        pl.BlockSpec((HP, KC), lambda i: (0, 0)),
        pl.BlockSpec((1, KC), lambda i: (0, 0)),
        pl.BlockSpec((IN_C, KC), lambda i: (0, 0)),
        pl.BlockSpec((KC, OUT_C), lambda i: (0, 0)),
    ],
    out_specs=pl.BlockSpec((BE, OUT_C), lambda i: (i, 0)),
    out_shape=jax.ShapeDtypeStruct((E_PAD, OUT_C), jnp.float32),
)


def _final_body(a0, a1, c0, c1, x_ref, rw, bias_ref, out_ref):
    cnt = jnp.maximum(c0[...] + c1[...], 1.0)
    aggr = (a0[...] + a1[...]) / cnt
    out_ref[...] = aggr + jnp.dot(x_ref[...], rw[...],
                                  preferred_element_type=jnp.float32) + bias_ref[...]


_final_call = pl.pallas_call(
    _final_body,
    out_shape=jax.ShapeDtypeStruct((N, OUT_C), jnp.float32),
)


def kernel(x, edge_index, edge_attr, W1, b1, W2, b2, W3, b3, W4, b4, root_w, bias):
    f32 = jnp.float32
    pad_e = E_PAD - E
    src = edge_index[0].astype(jnp.int32)
    dst = edge_index[1].astype(jnp.int32)
    src_p = jnp.concatenate([src, jnp.zeros((pad_e,), jnp.int32)]
                            ).reshape(E_PAD // CH, CH)
    dst_p = jnp.concatenate([dst, jnp.full((pad_e,), N, jnp.int32)]
                            ).reshape(E_PAD // CH, CH)
    ea_t = jnp.pad(edge_attr, ((0, pad_e), (0, 0))).T
    x_bf = x.astype(jnp.bfloat16)

    W1p = jnp.pad(W1, ((0, 0), (0, HP - H)))
    b1p = jnp.pad(b1, (0, HP - H)).reshape(1, HP)
    W2p = jnp.pad(W2, ((0, HP - H), (0, HP - H)))
    b2p = jnp.pad(b2, (0, HP - H)).reshape(1, HP)
    W3p = jnp.pad(W3, ((0, HP - H), (0, HP - H)))
    b3p = jnp.pad(b3, (0, HP - H)).reshape(1, HP)
    W4p = jnp.pad(W4, ((0, HP - H), (0, 0)))
    b4p = b4.reshape(1, KC)

    R = jnp.repeat(jnp.eye(IN_C, dtype=f32), OUT_C, axis=1)
    S = jnp.tile(jnp.eye(OUT_C, dtype=f32), (IN_C, 1))
    zeros_nb = jnp.zeros((N_PAD, OUT_C), f32)
    ones_ch = jnp.ones((CH, OUT_C), f32)

    x_j = _sc_gather(x_bf, src_p)
    msg = _mlp_call(ea_t, x_j, W1p, b1p, W2p, b2p, W3p, b3p, W4p, b4p, R, S)
    acc, cnt = _sc_scatter(msg, dst_p, zeros_nb, ones_ch)
    a0, a1 = acc[:N], acc[N_PAD:N_PAD + N]
    c0, c1 = cnt[:N], cnt[N_PAD:N_PAD + N]
    return _final_call(a0, a1, c0, c1, x, root_w, bias)


# speedup vs baseline: 1.5760x; 1.0113x over previous
import functools

import jax
import jax.numpy as jnp
from jax import lax
from jax.experimental import pallas as pl
from jax.experimental.pallas import tpu as pltpu
from jax.experimental.pallas import tpu_sc as plsc

N = 10000
E = 160000
IN_C = 16
OUT_C = 16
D_EDGE = 8
H = 100
HP = 128
KC = IN_C * OUT_C
LW = 128

NC = 2
NS = 16
NW = NC * NS
E_PAD = 163840
EPW = E_PAD // NW
CH = 128
NCHUNK = EPW // CH
N_PAD = 10240
RPW = N_PAD // NS

_sc_mesh = plsc.VectorSubcoreMesh(core_axis_name="c", subcore_axis_name="s")


@functools.partial(
    pl.kernel,
    mesh=_sc_mesh,
    out_type=jax.ShapeDtypeStruct((E_PAD, IN_C), jnp.bfloat16),
    scratch_types=[
        pltpu.VMEM((NCHUNK, CH), jnp.int32),
        pltpu.VMEM((EPW, IN_C), jnp.bfloat16),
        pltpu.SemaphoreType.DMA,
    ],
    compiler_params=pltpu.CompilerParams(use_tc_tiling_on_sc=False),
)
def _sc_gather(x_hbm, src_hbm, xj_hbm, idx_v, rows_v, sem):
    c = lax.axis_index("c")
    s = lax.axis_index("s")
    wid = c * NS + s
    pltpu.sync_copy(src_hbm.at[pl.ds(wid * NCHUNK, NCHUNK)], idx_v)

    def issue(j, carry):
        pltpu.async_copy(x_hbm.at[idx_v.at[j]], rows_v.at[pl.ds(j * CH, CH)], sem)
        return carry

    lax.fori_loop(0, NCHUNK, issue, 0)

    def drain(j, carry):
        pltpu.make_async_copy(
            x_hbm.at[idx_v.at[j]], rows_v.at[pl.ds(j * CH, CH)], sem
        ).wait()
        return carry

    lax.fori_loop(0, NCHUNK, drain, 0)
    pltpu.sync_copy(rows_v, xj_hbm.at[pl.ds(wid * EPW, EPW)])


@functools.partial(
    pl.kernel,
    mesh=_sc_mesh,
    out_type=(
        jax.ShapeDtypeStruct((NC * N_PAD, OUT_C), jnp.bfloat16),
        jax.ShapeDtypeStruct((NC * N_PAD, OUT_C), jnp.float32),
    ),
    scratch_types=[
        pltpu.VMEM((NCHUNK, CH), jnp.int32),
        pltpu.VMEM((EPW, OUT_C), jnp.bfloat16),
        pltpu.VMEM((CH, OUT_C), jnp.float32),
        pltpu.VMEM_SHARED((N_PAD, OUT_C), jnp.bfloat16),
        pltpu.VMEM_SHARED((N_PAD, OUT_C), jnp.float32),
    ],
    compiler_params=pltpu.CompilerParams(use_tc_tiling_on_sc=False),
)
def _sc_scatter(msg_hbm, dst_hbm, zeros_bf_hbm, zeros_hbm, ones_hbm,
                acc_out, cnt_out, idx_v, msg_v, ones_v, acc_sh, cnt_sh):
    c = lax.axis_index("c")
    s = lax.axis_index("s")
    wid = c * NS + s
    pltpu.sync_copy(dst_hbm.at[pl.ds(wid * NCHUNK, NCHUNK)], idx_v)
    pltpu.sync_copy(msg_hbm.at[pl.ds(wid * EPW, EPW)], msg_v)
    pltpu.sync_copy(ones_hbm, ones_v)

    @pl.when(s == 0)
    def _():
        pltpu.sync_copy(zeros_bf_hbm, acc_sh)
        pltpu.sync_copy(zeros_hbm, cnt_sh)

    plsc.subcore_barrier()

    def body(j, carry):
        row = idx_v.at[j]
        pltpu.sync_copy(msg_v.at[pl.ds(j * CH, CH)], acc_sh.at[row], add=True)
        pltpu.sync_copy(ones_v, cnt_sh.at[row], add=True)
        return carry

    lax.fori_loop(0, NCHUNK, body, 0)
    plsc.subcore_barrier()

    out_off = c * N_PAD + s * RPW
    pltpu.sync_copy(acc_sh.at[pl.ds(s * RPW, RPW)], acc_out.at[pl.ds(out_off, RPW)])
    pltpu.sync_copy(cnt_sh.at[pl.ds(s * RPW, RPW)], cnt_out.at[pl.ds(out_off, RPW)])


BE = 2048
GRID = E_PAD // BE


def _mlp_body(ea_ref, xj_ref, w1, b1, w2, b2, w3, b3, w4, b4, r_ref, s_ref,
              msg_ref):
    f32 = jnp.float32
    bf16 = jnp.bfloat16

    def mm(a, b):
        return jnp.dot(a.astype(bf16), b.astype(bf16), preferred_element_type=f32)

    h1 = lax.dot_general(ea_ref[...], w1[...].astype(bf16),
                         (((0,), (0,)), ((), ())),
                         preferred_element_type=f32)
    h = jnp.maximum(h1 + b1[...], 0.0)
    h = jnp.maximum(mm(h, w2[...]) + b2[...], 0.0)
    h = jnp.maximum(mm(h, w3[...]) + b3[...], 0.0)
    z = mm(h, w4[...]) + b4[...]
    xe = jnp.dot(xj_ref[...], r_ref[...].astype(bf16),
                 preferred_element_type=f32)
    msg_ref[...] = mm(z * xe, s_ref[...]).astype(bf16)


_mlp_call = pl.pallas_call(
    _mlp_body,
    grid=(GRID,),
    in_specs=[
        pl.BlockSpec((D_EDGE, BE), lambda i: (0, i)),
        pl.BlockSpec((BE, IN_C), lambda i: (i, 0)),
        pl.BlockSpec((D_EDGE, HP), lambda i: (0, 0)),
        pl.BlockSpec((1, HP), lambda i: (0, 0)),
        pl.BlockSpec((HP, HP), lambda i: (0, 0)),
        pl.BlockSpec((1, HP), lambda i: (0, 0)),
        pl.BlockSpec((HP, HP), lambda i: (0, 0)),
        pl.BlockSpec((1, HP), lambda i: (0, 0)),
        pl.BlockSpec((HP, KC), lambda i: (0, 0)),
        pl.BlockSpec((1, KC), lambda i: (0, 0)),
        pl.BlockSpec((IN_C, KC), lambda i: (0, 0)),
        pl.BlockSpec((KC, OUT_C), lambda i: (0, 0)),
    ],
    out_specs=pl.BlockSpec((BE, OUT_C), lambda i: (i, 0)),
    out_shape=jax.ShapeDtypeStruct((E_PAD, OUT_C), jnp.bfloat16),
)


def _final_body(a0, a1, c0, c1, x_ref, rw, bias_ref, out_ref):
    cnt = jnp.maximum(c0[...] + c1[...], 1.0)
    aggr = (a0[...].astype(jnp.float32) + a1[...].astype(jnp.float32)) / cnt
    out_ref[...] = aggr + jnp.dot(x_ref[...], rw[...],
                                  preferred_element_type=jnp.float32) + bias_ref[...]


_final_call = pl.pallas_call(
    _final_body,
    out_shape=jax.ShapeDtypeStruct((N, OUT_C), jnp.float32),
)


def kernel(x, edge_index, edge_attr, W1, b1, W2, b2, W3, b3, W4, b4, root_w, bias):
    f32 = jnp.float32
    pad_e = E_PAD - E
    src = edge_index[0].astype(jnp.int32)
    dst = edge_index[1].astype(jnp.int32)
    src_p = jnp.concatenate([src, jnp.zeros((pad_e,), jnp.int32)]
                            ).reshape(E_PAD // CH, CH)
    dst_p = jnp.concatenate([dst, jnp.full((pad_e,), N, jnp.int32)]
                            ).reshape(E_PAD // CH, CH)
    ea_t = jnp.pad(edge_attr, ((0, pad_e), (0, 0))).T.astype(jnp.bfloat16)
    x_bf = x.astype(jnp.bfloat16)

    W1p = jnp.pad(W1, ((0, 0), (0, HP - H)))
    b1p = jnp.pad(b1, (0, HP - H)).reshape(1, HP)
    W2p = jnp.pad(W2, ((0, HP - H), (0, HP - H)))
    b2p = jnp.pad(b2, (0, HP - H)).reshape(1, HP)
    W3p = jnp.pad(W3, ((0, HP - H), (0, HP - H)))
    b3p = jnp.pad(b3, (0, HP - H)).reshape(1, HP)
    W4p = jnp.pad(W4, ((0, HP - H), (0, 0)))
    b4p = b4.reshape(1, KC)

    R = jnp.repeat(jnp.eye(IN_C, dtype=f32), OUT_C, axis=1)
    S = jnp.tile(jnp.eye(OUT_C, dtype=f32), (IN_C, 1))
    zeros_nb = jnp.zeros((N_PAD, OUT_C), f32)
    zeros_bf = jnp.zeros((N_PAD, OUT_C), jnp.bfloat16)
    ones_ch = jnp.ones((CH, OUT_C), f32)

    x_j = _sc_gather(x_bf, src_p)
    msg = _mlp_call(ea_t, x_j, W1p, b1p, W2p, b2p, W3p, b3p, W4p, b4p, R, S)
    acc, cnt = _sc_scatter(msg, dst_p, zeros_bf, zeros_nb, ones_ch)
    a0, a1 = acc[:N], acc[N_PAD:N_PAD + N]
    c0, c1 = cnt[:N], cnt[N_PAD:N_PAD + N]
    return _final_call(a0, a1, c0, c1, x, root_w, bias)


# speedup vs baseline: 1.5827x; 1.0042x over previous
import functools

import jax
import jax.numpy as jnp
from jax import lax
from jax.experimental import pallas as pl
from jax.experimental.pallas import tpu as pltpu
from jax.experimental.pallas import tpu_sc as plsc

N = 10000
E = 160000
IN_C = 16
OUT_C = 16
D_EDGE = 8
H = 100
HP = 128
KC = IN_C * OUT_C
LW = 128

NC = 2
NS = 16
NW = NC * NS
E_PAD = 163840
EPW = E_PAD // NW
CH = 128
NCHUNK = EPW // CH
N_PAD = 10240
RPW = N_PAD // NS

_sc_mesh = plsc.VectorSubcoreMesh(core_axis_name="c", subcore_axis_name="s")


@functools.partial(
    pl.kernel,
    mesh=_sc_mesh,
    out_type=jax.ShapeDtypeStruct((E_PAD, IN_C), jnp.bfloat16),
    scratch_types=[
        pltpu.VMEM((NCHUNK, CH), jnp.int32),
        pltpu.VMEM((EPW, IN_C), jnp.bfloat16),
        pltpu.SemaphoreType.DMA,
    ],
    compiler_params=pltpu.CompilerParams(use_tc_tiling_on_sc=False),
)
def _sc_gather(x_hbm, src_hbm, xj_hbm, idx_v, rows_v, sem):
    c = lax.axis_index("c")
    s = lax.axis_index("s")
    wid = c * NS + s
    pltpu.sync_copy(src_hbm.at[pl.ds(wid * NCHUNK, NCHUNK)], idx_v)

    def issue(j, carry):
        pltpu.async_copy(x_hbm.at[idx_v.at[j]], rows_v.at[pl.ds(j * CH, CH)], sem)
        return carry

    lax.fori_loop(0, NCHUNK, issue, 0)

    def drain(j, carry):
        pltpu.make_async_copy(
            x_hbm.at[idx_v.at[j]], rows_v.at[pl.ds(j * CH, CH)], sem
        ).wait()
        return carry

    lax.fori_loop(0, NCHUNK, drain, 0)
    pltpu.sync_copy(rows_v, xj_hbm.at[pl.ds(wid * EPW, EPW)])


@functools.partial(
    pl.kernel,
    mesh=_sc_mesh,
    out_type=(
        jax.ShapeDtypeStruct((NC * N_PAD, OUT_C), jnp.float32),
        jax.ShapeDtypeStruct((NC * N_PAD, OUT_C), jnp.float32),
    ),
    scratch_types=[
        pltpu.VMEM((NCHUNK, CH), jnp.int32),
        pltpu.VMEM((IN_C, EPW), jnp.float32),
        pltpu.VMEM((CH, OUT_C), jnp.float32),
        pltpu.VMEM((CH, OUT_C), jnp.float32),
        pltpu.VMEM_SHARED((N_PAD, OUT_C), jnp.float32),
        pltpu.VMEM_SHARED((N_PAD, OUT_C), jnp.float32),
    ],
    compiler_params=pltpu.CompilerParams(use_tc_tiling_on_sc=False,
                                         needs_layout_passes=False),
)
def _sc_scatter(msg_hbm, dst_hbm, zeros_hbm, ones_hbm,
                acc_out, cnt_out, idx_v, msgt_v, msgf_v, ones_v, acc_sh, cnt_sh):
    c = lax.axis_index("c")
    s = lax.axis_index("s")
    wid = c * NS + s
    base = wid * EPW
    pltpu.sync_copy(dst_hbm.at[pl.ds(wid * NCHUNK, NCHUNK)], idx_v)
    for q in range(IN_C):
        pltpu.sync_copy(msg_hbm.at[q].at[pl.ds(base, EPW)], msgt_v.at[q])
    pltpu.sync_copy(ones_hbm, ones_v)

    @pl.when(s == 0)
    def _():
        pltpu.sync_copy(zeros_hbm, acc_sh)
        pltpu.sync_copy(zeros_hbm, cnt_sh)

    plsc.subcore_barrier()
    chan_ix = lax.iota(jnp.int32, 16)

    def body(j, carry):
        def edge(r, carry2):
            col = jnp.full((16,), j * CH + r, jnp.int32)
            msgf_v[r, :] = plsc.load_gather(msgt_v, [chan_ix, col])
            return carry2

        lax.fori_loop(0, CH, edge, 0)
        row = idx_v.at[j]
        pltpu.sync_copy(msgf_v, acc_sh.at[row], add=True)
        pltpu.sync_copy(ones_v, cnt_sh.at[row], add=True)
        return carry

    lax.fori_loop(0, NCHUNK, body, 0)
    plsc.subcore_barrier()

    out_off = c * N_PAD + s * RPW
    pltpu.sync_copy(acc_sh.at[pl.ds(s * RPW, RPW)], acc_out.at[pl.ds(out_off, RPW)])
    pltpu.sync_copy(cnt_sh.at[pl.ds(s * RPW, RPW)], cnt_out.at[pl.ds(out_off, RPW)])


BE = 2048
GRID = E_PAD // BE


def _mlp_body(ea_ref, xj_ref, w1, b1, w2, b2, w3, b3, w4, b4, r_ref, s_ref,
              msg_ref):
    f32 = jnp.float32
    bf16 = jnp.bfloat16

    def mm(a, b):
        return jnp.dot(a.astype(bf16), b.astype(bf16), preferred_element_type=f32)

    h1 = lax.dot_general(ea_ref[...], w1[...].astype(bf16),
                         (((0,), (0,)), ((), ())),
                         preferred_element_type=f32)
    h = jnp.maximum(h1 + b1[...], 0.0)
    h = jnp.maximum(mm(h, w2[...]) + b2[...], 0.0)
    h = jnp.maximum(mm(h, w3[...]) + b3[...], 0.0)
    z = mm(h, w4[...]) + b4[...]
    xe = jnp.dot(xj_ref[...], r_ref[...].astype(bf16),
                 preferred_element_type=f32)
    p = (z * xe).astype(bf16)
    msg_ref[...] = lax.dot_general(s_ref[...].astype(bf16), p,
                                   (((0,), (1,)), ((), ())),
                                   preferred_element_type=f32)


_mlp_call = pl.pallas_call(
    _mlp_body,
    grid=(GRID,),
    in_specs=[
        pl.BlockSpec((D_EDGE, BE), lambda i: (0, i)),
        pl.BlockSpec((BE, IN_C), lambda i: (i, 0)),
        pl.BlockSpec((D_EDGE, HP), lambda i: (0, 0)),
        pl.BlockSpec((1, HP), lambda i: (0, 0)),
        pl.BlockSpec((HP, HP), lambda i: (0, 0)),
        pl.BlockSpec((1, HP), lambda i: (0, 0)),
        pl.BlockSpec((HP, HP), lambda i: (0, 0)),
        pl.BlockSpec((1, HP), lambda i: (0, 0)),
        pl.BlockSpec((HP, KC), lambda i: (0, 0)),
        pl.BlockSpec((1, KC), lambda i: (0, 0)),
        pl.BlockSpec((IN_C, KC), lambda i: (0, 0)),
        pl.BlockSpec((KC, OUT_C), lambda i: (0, 0)),
    ],
    out_specs=pl.BlockSpec((OUT_C, BE), lambda i: (0, i)),
    out_shape=jax.ShapeDtypeStruct((OUT_C, E_PAD), jnp.float32),
)


def _final_body(a0, a1, c0, c1, x_ref, rw, bias_ref, out_ref):
    cnt = jnp.maximum(c0[...] + c1[...], 1.0)
    aggr = (a0[...] + a1[...]) / cnt
    out_ref[...] = aggr + jnp.dot(x_ref[...], rw[...],
                                  preferred_element_type=jnp.float32) + bias_ref[...]


_final_call = pl.pallas_call(
    _final_body,
    out_shape=jax.ShapeDtypeStruct((N, OUT_C), jnp.float32),
)


def kernel(x, edge_index, edge_attr, W1, b1, W2, b2, W3, b3, W4, b4, root_w, bias):
    f32 = jnp.float32
    pad_e = E_PAD - E
    src = edge_index[0].astype(jnp.int32)
    dst = edge_index[1].astype(jnp.int32)
    src_p = jnp.concatenate([src, jnp.zeros((pad_e,), jnp.int32)]
                            ).reshape(E_PAD // CH, CH)
    dst_p = jnp.concatenate([dst, jnp.full((pad_e,), N, jnp.int32)]
                            ).reshape(E_PAD // CH, CH)
    ea_t = jnp.pad(edge_attr, ((0, pad_e), (0, 0))).T.astype(jnp.bfloat16)
    x_bf = x.astype(jnp.bfloat16)

    W1p = jnp.pad(W1, ((0, 0), (0, HP - H)))
    b1p = jnp.pad(b1, (0, HP - H)).reshape(1, HP)
    W2p = jnp.pad(W2, ((0, HP - H), (0, HP - H)))
    b2p = jnp.pad(b2, (0, HP - H)).reshape(1, HP)
    W3p = jnp.pad(W3, ((0, HP - H), (0, HP - H)))
    b3p = jnp.pad(b3, (0, HP - H)).reshape(1, HP)
    W4p = jnp.pad(W4, ((0, HP - H), (0, 0)))
    b4p = b4.reshape(1, KC)

    R = jnp.repeat(jnp.eye(IN_C, dtype=f32), OUT_C, axis=1)
    S = jnp.tile(jnp.eye(OUT_C, dtype=f32), (IN_C, 1))
    zeros_nb = jnp.zeros((N_PAD, OUT_C), f32)
    ones_ch = jnp.ones((CH, OUT_C), f32)

    x_j = _sc_gather(x_bf, src_p)
    msg = _mlp_call(ea_t, x_j, W1p, b1p, W2p, b2p, W3p, b3p, W4p, b4p, R, S)
    acc, cnt = _sc_scatter(msg, dst_p, zeros_nb, ones_ch)
    a0, a1 = acc[:N], acc[N_PAD:N_PAD + N]
    c0, c1 = cnt[:N], cnt[N_PAD:N_PAD + N]
    return _final_call(a0, a1, c0, c1, x, root_w, bias)
